# baseline (speedup 1.0000x reference)
import jax
import jax.numpy as jnp
from jax import lax
from jax.experimental import pallas as pl
from jax.experimental.pallas import tpu as pltpu
from jax.experimental.pallas import tpu_sc as plsc

N = 10000
E = 320000
D_IN = 128
D_HID = 256
D_OUT = 128

NC, NS, LB = 2, 16, 128
NW = NC * NS
K = -(-E // (NW * LB))
EPAD = NW * K * LB
KT = 2 * K
K0 = 56
K1 = KT - K0
KMAX = max(K0, K1)
NPAD = 10240
R16 = NPAD // NS
BM = 1280
GRID = NPAD // BM

_mesh = plsc.VectorSubcoreMesh(
    core_axis_name="c", subcore_axis_name="s", num_cores=NC, num_subcores=NS)


def _sc_deg_body(dst_hbm, ones_hbm, zeros_hbm, out_hbm, didx_v, ones_v, acc_sh):
    c = lax.axis_index("c")
    s = lax.axis_index("s")
    wid = s * NC + c
    pltpu.sync_copy(zeros_hbm.at[pl.ds(s * R16, R16)],
                    acc_sh.at[pl.ds(s * R16, R16)])
    pltpu.sync_copy(dst_hbm.at[wid], didx_v)
    pltpu.sync_copy(ones_hbm, ones_v)
    plsc.subcore_barrier()

    def step(j, carry):
        pltpu.sync_copy(ones_v, acc_sh.at[didx_v.at[j]], add=True)
        return carry
    lax.fori_loop(0, K, step, 0)

    plsc.subcore_barrier()
    pltpu.sync_copy(acc_sh.at[pl.ds(s * R16, R16)],
                    out_hbm.at[c, pl.ds(s * R16, R16)])


_sc_deg = pl.kernel(
    _sc_deg_body,
    out_type=jax.ShapeDtypeStruct((NC, NPAD, 128), jnp.float32),
    mesh=_mesh,
    scratch_types=[
        pltpu.VMEM((K, LB), jnp.int32),
        pltpu.VMEM((LB, 128), jnp.float32),
        pltpu.VMEM_SHARED((NPAD, 128), jnp.float32),
    ],
)


def _sc_scatter_body(src0_hbm, dst0_hbm, src1_hbm, dst1_hbm, rows_hbm,
                     zeros_hbm, out_hbm, sidx_v, didx_v, bufa_v, acc_sh):
    c = lax.axis_index("c")
    s = lax.axis_index("s")

    @pl.when(c == 0)
    def _():
        pltpu.sync_copy(rows_hbm.at[pl.ds(s * R16, R16)],
                        acc_sh.at[pl.ds(s * R16, R16)])

    @pl.when(c == 1)
    def _():
        pltpu.sync_copy(zeros_hbm.at[pl.ds(s * R16, R16)],
                        acc_sh.at[pl.ds(s * R16, R16)])

    plsc.subcore_barrier()

    def step(j, carry):
        pltpu.sync_copy(rows_hbm.at[sidx_v.at[j]], bufa_v)
        pltpu.sync_copy(bufa_v, acc_sh.at[didx_v.at[j]], add=True)
        return carry

    @pl.when(c == 0)
    def _():
        pltpu.sync_copy(src0_hbm.at[s], sidx_v.at[pl.ds(0, K0)])
        pltpu.sync_copy(dst0_hbm.at[s], didx_v.at[pl.ds(0, K0)])
        lax.fori_loop(0, K0, step, 0)

    @pl.when(c == 1)
    def _():
        pltpu.sync_copy(src1_hbm.at[s], sidx_v.at[pl.ds(0, K1)])
        pltpu.sync_copy(dst1_hbm.at[s], didx_v.at[pl.ds(0, K1)])
        lax.fori_loop(0, K1, step, 0)

    plsc.subcore_barrier()
    pltpu.sync_copy(acc_sh.at[pl.ds(s * R16, R16)],
                    out_hbm.at[c, pl.ds(s * R16, R16)])


_sc_scatter = pl.kernel(
    _sc_scatter_body,
    out_type=jax.ShapeDtypeStruct((NC, NPAD, 128), jnp.float32),
    mesh=_mesh,
    scratch_types=[
        pltpu.VMEM((KMAX, LB), jnp.int32),
        pltpu.VMEM((KMAX, LB), jnp.int32),
        pltpu.VMEM((LB, 128), jnp.float32),
        pltpu.VMEM_SHARED((NPAD, 128), jnp.float32),
    ],
)


def _tc_prep_body(c0_ref, c1_ref, x_ref, xs_ref, dv_ref):
    deg = c0_ref[0] + c1_ref[0] + 1.0
    dv = lax.rsqrt(deg)
    dv_ref[...] = dv[:, 0:16]
    xs_ref[...] = x_ref[...] * dv


_tc_prep = pl.pallas_call(
    _tc_prep_body,
    grid=(GRID,),
    in_specs=[
        pl.BlockSpec((1, BM, 128), lambda i: (0, i, 0)),
        pl.BlockSpec((1, BM, 128), lambda i: (1, i, 0)),
        pl.BlockSpec((BM, 128), lambda i: (i, 0)),
    ],
    out_specs=[
        pl.BlockSpec((BM, 128), lambda i: (i, 0)),
        pl.BlockSpec((BM, 16), lambda i: (i, 0)),
    ],
    out_shape=[
        jax.ShapeDtypeStruct((NPAD, 128), jnp.float32),
        jax.ShapeDtypeStruct((NPAD, 16), jnp.float32),
    ],
)


def _tc_mid_body(p0_ref, p1_ref, dv_ref, w1_ref, b1_ref, w2_ref, o_ref):
    d = dv_ref[...][:, 0:1]
    z = (p0_ref[0] + p1_ref[0]) * d
    h = jnp.dot(z, w1_ref[...], preferred_element_type=jnp.float32,
                precision=lax.Precision.HIGHEST)
    h = jnp.maximum(h + b1_ref[...], 0.0)
    y = jnp.dot(h, w2_ref[...], preferred_element_type=jnp.float32,
                precision=lax.Precision.HIGHEST)
    o_ref[...] = y * d


_tc_mid = pl.pallas_call(
    _tc_mid_body,
    grid=(GRID,),
    in_specs=[
        pl.BlockSpec((1, BM, 128), lambda i: (0, i, 0)),
        pl.BlockSpec((1, BM, 128), lambda i: (1, i, 0)),
        pl.BlockSpec((BM, 16), lambda i: (i, 0)),
        pl.BlockSpec((D_IN, D_HID), lambda i: (0, 0)),
        pl.BlockSpec((1, D_HID), lambda i: (0, 0)),
        pl.BlockSpec((D_HID, D_OUT), lambda i: (0, 0)),
    ],
    out_specs=pl.BlockSpec((BM, 128), lambda i: (i, 0)),
    out_shape=jax.ShapeDtypeStruct((NPAD, 128), jnp.float32),
)


def _tc_fin_body(q0_ref, q1_ref, dv_ref, b2_ref, o_ref):
    d = dv_ref[...][:, 0:1]
    o_ref[...] = jnp.maximum((q0_ref[0] + q1_ref[0]) * d + b2_ref[...], 0.0)


_tc_fin = pl.pallas_call(
    _tc_fin_body,
    grid=(GRID,),
    in_specs=[
        pl.BlockSpec((1, BM, 128), lambda i: (0, i, 0)),
        pl.BlockSpec((1, BM, 128), lambda i: (1, i, 0)),
        pl.BlockSpec((BM, 16), lambda i: (i, 0)),
        pl.BlockSpec((1, D_OUT), lambda i: (0, 0)),
    ],
    out_specs=pl.BlockSpec((BM, 128), lambda i: (i, 0)),
    out_shape=jax.ShapeDtypeStruct((NPAD, 128), jnp.float32),
)


def kernel(edge_index, edge_weight, W1, b1, W2, b2):
    ei = edge_index.astype(jnp.int32)
    pad_e = EPAD - E
    src = jnp.concatenate([ei[0], jnp.zeros((pad_e,), jnp.int32)]).reshape(NS, KT, LB)
    dst = jnp.concatenate([ei[1], jnp.full((pad_e,), N, jnp.int32)]).reshape(NS, KT, LB)
    src0, src1 = src[:, :K0], src[:, K0:]
    dst0, dst1 = dst[:, :K0], dst[:, K0:]
    dst_bal = dst.reshape(NW, K, LB)
    x_pad = jnp.pad(edge_weight, ((0, NPAD - N), (0, 0)))
    zeros_rows = jnp.zeros((NPAD, 128), jnp.float32)
    ones_rows = jnp.ones((LB, 128), jnp.float32)

    cnt = _sc_deg(dst_bal, ones_rows, zeros_rows)
    xs, dv = _tc_prep(cnt, cnt, x_pad)
    p = _sc_scatter(src0, dst0, src1, dst1, xs, zeros_rows)
    ys2 = _tc_mid(p, p, dv, W1, b1.reshape(1, -1), W2)
    q = _sc_scatter(src0, dst0, src1, dst1, ys2, zeros_rows)
    out = _tc_fin(q, q, dv, b2.reshape(1, -1))
    return out[:N]


# speedup vs baseline: 1.1003x; 1.1003x over previous
import jax
import jax.numpy as jnp
from jax import lax
from jax.experimental import pallas as pl
from jax.experimental.pallas import tpu as pltpu
from jax.experimental.pallas import tpu_sc as plsc

N = 10000
E = 320000
D_IN = 128
D_HID = 256
D_OUT = 128

NC, NS, LB = 2, 16, 128
NW = NC * NS
K = -(-E // (NW * LB))
EPAD = NW * K * LB
KT = 2 * K
K0 = K
K1 = KT - K0
KMAX = max(K0, K1)
NPAD = 10240
R16 = NPAD // NS
BM = 1280
GRID = NPAD // BM

_mesh = plsc.VectorSubcoreMesh(
    core_axis_name="c", subcore_axis_name="s", num_cores=NC, num_subcores=NS)


def _sc_deg_body(dst_hbm, ones_hbm, zeros_hbm, out_hbm, didx_v, ones_v, acc_sh):
    c = lax.axis_index("c")
    s = lax.axis_index("s")
    wid = s * NC + c
    pltpu.sync_copy(zeros_hbm.at[pl.ds(s * R16, R16)],
                    acc_sh.at[pl.ds(s * R16, R16)])
    pltpu.sync_copy(dst_hbm.at[wid], didx_v)
    pltpu.sync_copy(ones_hbm, ones_v)
    plsc.subcore_barrier()

    def step(j, carry):
        pltpu.sync_copy(ones_v, acc_sh.at[didx_v.at[j]], add=True)
        return carry
    lax.fori_loop(0, K, step, 0)

    plsc.subcore_barrier()
    pltpu.sync_copy(acc_sh.at[pl.ds(s * R16, R16)],
                    out_hbm.at[c, pl.ds(s * R16, R16)])


_sc_deg = pl.kernel(
    _sc_deg_body,
    out_type=jax.ShapeDtypeStruct((NC, NPAD, 128), jnp.float32),
    mesh=_mesh,
    scratch_types=[
        pltpu.VMEM((K, LB), jnp.int32),
        pltpu.VMEM((LB, 128), jnp.float32),
        pltpu.VMEM_SHARED((NPAD, 128), jnp.float32),
    ],
)


def _sc_scatter_body(src0_hbm, dst0_hbm, src1_hbm, dst1_hbm, rows_hbm,
                     zeros_hbm, out_hbm, sidx_v, didx_v, bufa_v, acc_sh):
    c = lax.axis_index("c")
    s = lax.axis_index("s")

    @pl.when(c == 0)
    def _():
        pltpu.sync_copy(rows_hbm.at[pl.ds(s * R16, R16)],
                        acc_sh.at[pl.ds(s * R16, R16)])

    @pl.when(c == 1)
    def _():
        pltpu.sync_copy(zeros_hbm.at[pl.ds(s * R16, R16)],
                        acc_sh.at[pl.ds(s * R16, R16)])

    plsc.subcore_barrier()

    def step(j, carry):
        pltpu.sync_copy(rows_hbm.at[sidx_v.at[j]], bufa_v)
        pltpu.sync_copy(bufa_v, acc_sh.at[didx_v.at[j]], add=True)
        return carry

    @pl.when(c == 0)
    def _():
        pltpu.sync_copy(src0_hbm.at[s], sidx_v.at[pl.ds(0, K0)])
        pltpu.sync_copy(dst0_hbm.at[s], didx_v.at[pl.ds(0, K0)])
        lax.fori_loop(0, K0, step, 0)

    @pl.when(c == 1)
    def _():
        pltpu.sync_copy(src1_hbm.at[s], sidx_v.at[pl.ds(0, K1)])
        pltpu.sync_copy(dst1_hbm.at[s], didx_v.at[pl.ds(0, K1)])
        lax.fori_loop(0, K1, step, 0)

    plsc.subcore_barrier()
    pltpu.sync_copy(acc_sh.at[pl.ds(s * R16, R16)],
                    out_hbm.at[c, pl.ds(s * R16, R16)])


_sc_scatter = pl.kernel(
    _sc_scatter_body,
    out_type=jax.ShapeDtypeStruct((NC, NPAD, 128), jnp.float32),
    mesh=_mesh,
    scratch_types=[
        pltpu.VMEM((KMAX, LB), jnp.int32),
        pltpu.VMEM((KMAX, LB), jnp.int32),
        pltpu.VMEM((LB, 128), jnp.float32),
        pltpu.VMEM_SHARED((NPAD, 128), jnp.float32),
    ],
)


def _tc_prep_body(c0_ref, c1_ref, x_ref, xs_ref, dv_ref):
    deg = c0_ref[0] + c1_ref[0] + 1.0
    dv = lax.rsqrt(deg)
    dv_ref[...] = dv[:, 0:16]
    xs_ref[...] = x_ref[...] * dv


_tc_prep = pl.pallas_call(
    _tc_prep_body,
    grid=(GRID,),
    in_specs=[
        pl.BlockSpec((1, BM, 128), lambda i: (0, i, 0)),
        pl.BlockSpec((1, BM, 128), lambda i: (1, i, 0)),
        pl.BlockSpec((BM, 128), lambda i: (i, 0)),
    ],
    out_specs=[
        pl.BlockSpec((BM, 128), lambda i: (i, 0)),
        pl.BlockSpec((BM, 16), lambda i: (i, 0)),
    ],
    out_shape=[
        jax.ShapeDtypeStruct((NPAD, 128), jnp.float32),
        jax.ShapeDtypeStruct((NPAD, 16), jnp.float32),
    ],
)


def _tc_mid_body(p0_ref, p1_ref, dv_ref, w1_ref, b1_ref, w2_ref, o_ref):
    d = dv_ref[...][:, 0:1]
    z = (p0_ref[0] + p1_ref[0]) * d
    h = jnp.dot(z, w1_ref[...], preferred_element_type=jnp.float32)
    h = jnp.maximum(h + b1_ref[...], 0.0)
    y = jnp.dot(h, w2_ref[...], preferred_element_type=jnp.float32)
    o_ref[...] = y * d


_tc_mid = pl.pallas_call(
    _tc_mid_body,
    grid=(GRID,),
    in_specs=[
        pl.BlockSpec((1, BM, 128), lambda i: (0, i, 0)),
        pl.BlockSpec((1, BM, 128), lambda i: (1, i, 0)),
        pl.BlockSpec((BM, 16), lambda i: (i, 0)),
        pl.BlockSpec((D_IN, D_HID), lambda i: (0, 0)),
        pl.BlockSpec((1, D_HID), lambda i: (0, 0)),
        pl.BlockSpec((D_HID, D_OUT), lambda i: (0, 0)),
    ],
    out_specs=pl.BlockSpec((BM, 128), lambda i: (i, 0)),
    out_shape=jax.ShapeDtypeStruct((NPAD, 128), jnp.float32),
)


def _tc_fin_body(q0_ref, q1_ref, dv_ref, b2_ref, o_ref):
    d = dv_ref[...][:, 0:1]
    o_ref[...] = jnp.maximum((q0_ref[0] + q1_ref[0]) * d + b2_ref[...], 0.0)


_tc_fin = pl.pallas_call(
    _tc_fin_body,
    grid=(GRID,),
    in_specs=[
        pl.BlockSpec((1, BM, 128), lambda i: (0, i, 0)),
        pl.BlockSpec((1, BM, 128), lambda i: (1, i, 0)),
        pl.BlockSpec((BM, 16), lambda i: (i, 0)),
        pl.BlockSpec((1, D_OUT), lambda i: (0, 0)),
    ],
    out_specs=pl.BlockSpec((BM, 128), lambda i: (i, 0)),
    out_shape=jax.ShapeDtypeStruct((NPAD, 128), jnp.float32),
)


def kernel(edge_index, edge_weight, W1, b1, W2, b2):
    ei = edge_index.astype(jnp.int32)
    pad_e = EPAD - E
    src = jnp.concatenate([ei[0], jnp.zeros((pad_e,), jnp.int32)]).reshape(NS, KT, LB)
    dst = jnp.concatenate([ei[1], jnp.full((pad_e,), N, jnp.int32)]).reshape(NS, KT, LB)
    src0, src1 = src[:, :K0], src[:, K0:]
    dst0, dst1 = dst[:, :K0], dst[:, K0:]
    dst_bal = dst.reshape(NW, K, LB)
    x_pad = jnp.pad(edge_weight, ((0, NPAD - N), (0, 0)))
    zeros_rows = jnp.zeros((NPAD, 128), jnp.float32)
    ones_rows = jnp.ones((LB, 128), jnp.float32)

    cnt = _sc_deg(dst_bal, ones_rows, zeros_rows)
    xs, dv = _tc_prep(cnt, cnt, x_pad)
    p = _sc_scatter(src0, dst0, src1, dst1, xs, zeros_rows)
    ys2 = _tc_mid(p, p, dv, W1, b1.reshape(1, -1), W2)
    q = _sc_scatter(src0, dst0, src1, dst1, ys2, zeros_rows)
    out = _tc_fin(q, q, dv, b2.reshape(1, -1))
    return out[:N]
